# baseline (speedup 1.0000x reference)
import functools
import math

import numpy as np
import jax
import jax.numpy as jnp
from jax import lax
from jax.experimental import pallas as pl
from jax.experimental.pallas import tpu as pltpu
from jax.experimental.pallas import tpu_sc as plsc

_VOCAB = 100000
_D = 1024
_B = 4
_L = 2048
_NC, _NS = 2, 16
_NW = _NC * _NS
_PPW = _L // _NW
_ROWS = _B * _L
_CHUNK = 16
_NGROUP = _PPW // _CHUNK
_NCHUNK = _NGROUP * _B
_NROT = _L // _CHUNK
_LANES = 16
_VPC = _CHUNK * _D // _LANES


def _pe_consts() -> np.ndarray:
    div = np.exp(np.arange(0, _D, 2).astype(np.float32)
                 * (-math.log(10000.0) / _D))
    pos = np.arange(_CHUNK, dtype=np.float32)[:, None]
    pe16 = np.zeros((_CHUNK, _D), dtype=np.float32)
    pe16[:, 0::2] = np.sin(pos * div)
    pe16[:, 1::2] = np.cos(pos * div)
    pe16s = np.zeros_like(pe16)
    pe16s[:, 0::2] = pe16[:, 1::2]
    pe16s[:, 1::2] = pe16[:, 0::2]
    k = (np.arange(_NROT, dtype=np.float32) * _CHUNK)[:, None]
    rot_a = np.zeros((_NROT, _D), dtype=np.float32)
    rot_a[:, 0::2] = np.cos(k * div)
    rot_a[:, 1::2] = rot_a[:, 0::2]
    rot_b = np.zeros((_NROT, _D), dtype=np.float32)
    rot_b[:, 0::2] = np.sin(k * div)
    rot_b[:, 1::2] = -rot_b[:, 0::2]
    return np.concatenate([pe16, pe16s, rot_a, rot_b], axis=0)


_CONSTS = _pe_consts()


_NBUF = 4


def _embed_body(x_hbm, consts_hbm, table_hbm, out_hbm,
                idx_v, pe16, pe16s, rota, rotb, pe_v, gb0, gb1, gb2, gb3,
                gsem0, gsem1, gsem2, gsem3, ssem0, ssem1, ssem2, ssem3):
    gbuf = (gb0, gb1, gb2, gb3)
    gsem = (gsem0, gsem1, gsem2, gsem3)
    ssem = (ssem0, ssem1, ssem2, ssem3)

    wid = lax.axis_index("s") * _NC + lax.axis_index("c")
    p0 = wid * _PPW
    k0 = wid * _NGROUP

    for b in range(_B):
        pltpu.sync_copy(x_hbm.at[b, pl.ds(p0, _PPW)],
                        idx_v.at[pl.ds(b * _PPW, _PPW)])

    def start_gather(c):
        g, bt = divmod(c, _B)
        return pltpu.async_copy(
            table_hbm.at[idx_v.at[pl.ds(bt * _PPW + g * _CHUNK, _CHUNK)]],
            gbuf[c % _NBUF], gsem[c % _NBUF])

    gathers = {c: start_gather(c) for c in range(_NBUF - 1)}
    stores = {}

    pltpu.sync_copy(consts_hbm.at[pl.ds(0, _CHUNK)], pe16)
    pltpu.sync_copy(consts_hbm.at[pl.ds(_CHUNK, _CHUNK)], pe16s)
    pltpu.sync_copy(consts_hbm.at[pl.ds(2 * _CHUNK + k0, _NGROUP)], rota)
    pltpu.sync_copy(
        consts_hbm.at[pl.ds(2 * _CHUNK + _NROT + k0, _NGROUP)], rotb)

    def build_pe(g):
        @plsc.parallel_loop(0, _D // _LANES, 1, unroll=2)
        def col_body(j):
            s = pl.ds(pl.multiple_of(j * _LANES, _LANES), _LANES)
            va = rota[g, s]
            vb = rotb[g, s]

            @plsc.parallel_loop(0, _CHUNK, 1, unroll=16)
            def row_body(r):
                pe_v[r, s] = pe16[r, s] * va + pe16s[r, s] * vb

    build_pe(0)

    for c in range(_NCHUNK):
        g, bt = divmod(c, _B)
        if bt == 0 and g > 0:
            build_pe(g)
        gathers.pop(c).wait()

        gb = gbuf[c % _NBUF]

        @plsc.parallel_loop(0, _D // _LANES, 1, unroll=2)
        def add_col(j):
            s = pl.ds(pl.multiple_of(j * _LANES, _LANES), _LANES)

            @plsc.parallel_loop(0, _CHUNK, 1, unroll=16)
            def add_row(r):
                plsc.addupdate(gb.at[r, s], pe_v[r, s])

        stores[c] = pltpu.async_copy(
            gb, out_hbm.at[pl.ds(bt * _L + p0 + g * _CHUNK, _CHUNK)],
            ssem[c % _NBUF])
        if c + _NBUF - 1 < _NCHUNK:
            if c >= 1:
                stores.pop(c - 1).wait()
            gathers[c + _NBUF - 1] = start_gather(c + _NBUF - 1)

    for c in sorted(stores):
        stores.pop(c).wait()


@jax.jit
def _embed(x, consts, table):
    mesh = plsc.VectorSubcoreMesh(core_axis_name="c", subcore_axis_name="s")
    f = functools.partial(
        pl.kernel,
        mesh=mesh,
        out_type=jax.ShapeDtypeStruct((_ROWS, _D), jnp.float32),
        scratch_types=[
            pltpu.VMEM((_B * _PPW,), jnp.int32),
            pltpu.VMEM((_CHUNK, _D), jnp.float32),
            pltpu.VMEM((_CHUNK, _D), jnp.float32),
            pltpu.VMEM((_NGROUP, _D), jnp.float32),
            pltpu.VMEM((_NGROUP, _D), jnp.float32),
            pltpu.VMEM((_CHUNK, _D), jnp.float32),
            pltpu.VMEM((_CHUNK, _D), jnp.float32),
            pltpu.VMEM((_CHUNK, _D), jnp.float32),
            pltpu.VMEM((_CHUNK, _D), jnp.float32),
            pltpu.VMEM((_CHUNK, _D), jnp.float32),
            pltpu.SemaphoreType.DMA,
            pltpu.SemaphoreType.DMA,
            pltpu.SemaphoreType.DMA,
            pltpu.SemaphoreType.DMA,
            pltpu.SemaphoreType.DMA,
            pltpu.SemaphoreType.DMA,
            pltpu.SemaphoreType.DMA,
            pltpu.SemaphoreType.DMA,
        ],
    )(_embed_body)
    return f(x, consts, table)


def kernel(x, table):
    consts = jnp.asarray(_CONSTS)
    out = _embed(x, consts, table)
    return out.reshape(_B, _L, _D)


# speedup vs baseline: 1.0008x; 1.0008x over previous
import functools
import math

import numpy as np
import jax
import jax.numpy as jnp
from jax import lax
from jax.experimental import pallas as pl
from jax.experimental.pallas import tpu as pltpu
from jax.experimental.pallas import tpu_sc as plsc

_VOCAB = 100000
_D = 1024
_B = 4
_L = 2048
_NC, _NS = 2, 16
_NW = _NC * _NS
_PPW = _L // _NW
_ROWS = _B * _L
_CHUNK = 16
_NGROUP = _PPW // _CHUNK
_NCHUNK = _NGROUP * _B
_NROT = _L // _CHUNK
_LANES = 16
_VPC = _CHUNK * _D // _LANES


def _pe_consts() -> np.ndarray:
    div = np.exp(np.arange(0, _D, 2).astype(np.float32)
                 * (-math.log(10000.0) / _D))
    pos = np.arange(_CHUNK, dtype=np.float32)[:, None]
    pe16 = np.zeros((_CHUNK, _D), dtype=np.float32)
    pe16[:, 0::2] = np.sin(pos * div)
    pe16[:, 1::2] = np.cos(pos * div)
    pe16s = np.zeros_like(pe16)
    pe16s[:, 0::2] = pe16[:, 1::2]
    pe16s[:, 1::2] = pe16[:, 0::2]
    k = (np.arange(_NROT, dtype=np.float32) * _CHUNK)[:, None]
    rot_a = np.zeros((_NROT, _D), dtype=np.float32)
    rot_a[:, 0::2] = np.cos(k * div)
    rot_a[:, 1::2] = rot_a[:, 0::2]
    rot_b = np.zeros((_NROT, _D), dtype=np.float32)
    rot_b[:, 0::2] = np.sin(k * div)
    rot_b[:, 1::2] = -rot_b[:, 0::2]
    return np.concatenate([pe16, pe16s, rot_a, rot_b], axis=0)


_CONSTS = _pe_consts()


def _embed_body(x_hbm, consts_hbm, table_hbm, out_hbm,
                idx_v, pe16, pe16s, rota, rotb, pe_v, gb0, gb1, ob0, ob1,
                gsem0, gsem1, ssem0, ssem1):
    gbuf = (gb0, gb1)
    obuf = (ob0, ob1)
    gsem = (gsem0, gsem1)
    ssem = (ssem0, ssem1)

    wid = lax.axis_index("s") * _NC + lax.axis_index("c")
    p0 = wid * _PPW
    k0 = wid * _NGROUP

    for b in range(_B):
        pltpu.sync_copy(x_hbm.at[b, pl.ds(p0, _PPW)],
                        idx_v.at[pl.ds(b * _PPW, _PPW)])

    def start_gather(c):
        g, bt = divmod(c, _B)
        return pltpu.async_copy(
            table_hbm.at[idx_v.at[pl.ds(bt * _PPW + g * _CHUNK, _CHUNK)]],
            gbuf[c % 2], gsem[c % 2])

    gathers = {0: start_gather(0), 1: start_gather(1)}
    stores = {}

    pltpu.sync_copy(consts_hbm.at[pl.ds(0, _CHUNK)], pe16)
    pltpu.sync_copy(consts_hbm.at[pl.ds(_CHUNK, _CHUNK)], pe16s)
    pltpu.sync_copy(consts_hbm.at[pl.ds(2 * _CHUNK + k0, _NGROUP)], rota)
    pltpu.sync_copy(
        consts_hbm.at[pl.ds(2 * _CHUNK + _NROT + k0, _NGROUP)], rotb)

    def build_pe(g):
        @plsc.parallel_loop(0, _D // (2 * _LANES), 1, unroll=2)
        def col_body(j):
            c0 = pl.multiple_of(j * 2 * _LANES, 2 * _LANES)
            s1 = pl.ds(c0, _LANES)
            s2 = pl.ds(c0 + _LANES, _LANES)
            va1, va2 = rota[g, s1], rota[g, s2]
            vb1, vb2 = rotb[g, s1], rotb[g, s2]

            @plsc.parallel_loop(0, _CHUNK, 1, unroll=8)
            def row_body(r):
                pe_v[r, s1] = pe16[r, s1] * va1 + pe16s[r, s1] * vb1
                pe_v[r, s2] = pe16[r, s2] * va2 + pe16s[r, s2] * vb2

    build_pe(0)

    for c in range(_NCHUNK):
        g, bt = divmod(c, _B)
        if bt == 0 and g > 0:
            build_pe(g)
        gathers.pop(c).wait()
        if c >= 2:
            stores.pop(c - 2).wait()

        gb, ob = gbuf[c % 2], obuf[c % 2]

        @plsc.parallel_loop(0, _VPC // 2, 1, unroll=4)
        def add_body(i):
            r = lax.shift_right_logical(i, 5)
            c0 = pl.multiple_of(
                lax.bitwise_and(i, _D // (2 * _LANES) - 1) * 2 * _LANES,
                2 * _LANES)
            s1 = pl.ds(c0, _LANES)
            s2 = pl.ds(c0 + _LANES, _LANES)
            ob[r, s1] = gb[r, s1] + pe_v[r, s1]
            ob[r, s2] = gb[r, s2] + pe_v[r, s2]

        stores[c] = pltpu.async_copy(
            ob, out_hbm.at[pl.ds(bt * _L + p0 + g * _CHUNK, _CHUNK)],
            ssem[c % 2])
        if c + 2 < _NCHUNK:
            gathers[c + 2] = start_gather(c + 2)

    stores.pop(_NCHUNK - 2).wait()
    stores.pop(_NCHUNK - 1).wait()


@jax.jit
def _embed(x, consts, table):
    mesh = plsc.VectorSubcoreMesh(core_axis_name="c", subcore_axis_name="s")
    f = functools.partial(
        pl.kernel,
        mesh=mesh,
        out_type=jax.ShapeDtypeStruct((_ROWS, _D), jnp.float32),
        scratch_types=[
            pltpu.VMEM((_B * _PPW,), jnp.int32),
            pltpu.VMEM((_CHUNK, _D), jnp.float32),
            pltpu.VMEM((_CHUNK, _D), jnp.float32),
            pltpu.VMEM((_NGROUP, _D), jnp.float32),
            pltpu.VMEM((_NGROUP, _D), jnp.float32),
            pltpu.VMEM((_CHUNK, _D), jnp.float32),
            pltpu.VMEM((_CHUNK, _D), jnp.float32),
            pltpu.VMEM((_CHUNK, _D), jnp.float32),
            pltpu.VMEM((_CHUNK, _D), jnp.float32),
            pltpu.VMEM((_CHUNK, _D), jnp.float32),
            pltpu.SemaphoreType.DMA,
            pltpu.SemaphoreType.DMA,
            pltpu.SemaphoreType.DMA,
            pltpu.SemaphoreType.DMA,
        ],
    )(_embed_body)
    return f(x, consts, table)


def kernel(x, table):
    consts = jnp.asarray(_CONSTS)
    out = _embed(x, consts, table)
    return out.reshape(_B, _L, _D)


# speedup vs baseline: 1.0990x; 1.0982x over previous
import functools
import math

import numpy as np
import jax
import jax.numpy as jnp
from jax import lax
from jax.experimental import pallas as pl
from jax.experimental.pallas import tpu as pltpu
from jax.experimental.pallas import tpu_sc as plsc

_VOCAB = 100000
_D = 1024
_B = 4
_L = 2048
_NC, _NS = 2, 16
_NW = _NC * _NS
_PPW = _L // _NW
_ROWS = _B * _L
_CHUNK = 16
_NGROUP = _PPW // _CHUNK
_NCHUNK = _NGROUP * _B
_LANES = 16
_NBUF = 4


_NROT = _L // _CHUNK


def _pe_consts() -> np.ndarray:
    div = np.exp(np.arange(0, _D, 2).astype(np.float32)
                 * (-math.log(10000.0) / _D))
    pos = np.arange(_CHUNK, dtype=np.float32)[:, None]
    pe16 = np.zeros((_CHUNK, _D), dtype=np.float32)
    pe16[:, 0::2] = np.sin(pos * div)
    pe16[:, 1::2] = np.cos(pos * div)
    pe16s = np.zeros_like(pe16)
    pe16s[:, 0::2] = pe16[:, 1::2]
    pe16s[:, 1::2] = pe16[:, 0::2]
    k = (np.arange(_NROT, dtype=np.float32) * _CHUNK)[:, None]
    rot_a = np.zeros((_NROT, _D), dtype=np.float32)
    rot_a[:, 0::2] = np.cos(k * div)
    rot_a[:, 1::2] = rot_a[:, 0::2]
    rot_b = np.zeros((_NROT, _D), dtype=np.float32)
    rot_b[:, 0::2] = np.sin(k * div)
    rot_b[:, 1::2] = -rot_b[:, 0::2]
    return np.concatenate([pe16, pe16s, rot_a, rot_b], axis=0)


_CONSTS = _pe_consts()


def _embed_body(x_hbm, pe_hbm, table_hbm, out_hbm,
                idx_v, pv0, pv1, gb0, gb1, gb2, gb3,
                psem0, psem1, gsem0, gsem1, gsem2, gsem3,
                ssem0, ssem1, ssem2, ssem3):
    pe_v = (pv0, pv1)
    gbuf = (gb0, gb1, gb2, gb3)
    psem = (psem0, psem1)
    gsem = (gsem0, gsem1, gsem2, gsem3)
    ssem = (ssem0, ssem1, ssem2, ssem3)

    wid = lax.axis_index("s") * _NC + lax.axis_index("c")
    p0 = wid * _PPW

    for b in range(_B):
        pltpu.sync_copy(x_hbm.at[b, pl.ds(p0, _PPW)],
                        idx_v.at[pl.ds(b * _PPW, _PPW)])

    def start_gather(c):
        g, bt = divmod(c, _B)
        return pltpu.async_copy(
            table_hbm.at[idx_v.at[pl.ds(bt * _PPW + g * _CHUNK, _CHUNK)]],
            gbuf[c % _NBUF], gsem[c % _NBUF])

    def start_pe(g):
        return pltpu.async_copy(
            pe_hbm.at[pl.ds(p0 + g * _CHUNK, _CHUNK)], pe_v[g % 2],
            psem[g % 2])

    gathers = {c: start_gather(c) for c in range(_NBUF - 1)}
    pes = {0: start_pe(0), 1: start_pe(1)}
    stores = {}

    for c in range(_NCHUNK):
        g, bt = divmod(c, _B)
        if bt == 0:
            if 1 <= g <= _NGROUP - 2:
                pes[g + 1] = start_pe(g + 1)
            pes.pop(g).wait()
        gathers.pop(c).wait()

        gb, pv = gbuf[c % _NBUF], pe_v[g % 2]

        @plsc.parallel_loop(0, _CHUNK * _D // (2 * _LANES), 1, unroll=4)
        def add_body(i):
            r = lax.shift_right_logical(i, 5)
            c0 = pl.multiple_of(
                lax.bitwise_and(i, _D // (2 * _LANES) - 1) * 2 * _LANES,
                2 * _LANES)
            s1 = pl.ds(c0, _LANES)
            s2 = pl.ds(c0 + _LANES, _LANES)
            plsc.addupdate(gb.at[r, s1], pv[r, s1])
            plsc.addupdate(gb.at[r, s2], pv[r, s2])

        stores[c] = pltpu.async_copy(
            gb, out_hbm.at[pl.ds(bt * _L + p0 + g * _CHUNK, _CHUNK)],
            ssem[c % _NBUF])
        if c + _NBUF - 1 < _NCHUNK:
            if c >= 1:
                stores.pop(c - 1).wait()
            gathers[c + _NBUF - 1] = start_gather(c + _NBUF - 1)

    for c in sorted(stores):
        stores.pop(c).wait()


@jax.jit
def _embed(x, table):
    anchor = jnp.float32(0) * table[0, 0]
    consts = jnp.asarray(_CONSTS)
    pe16 = consts[0:_CHUNK][None] + anchor
    pe16s = consts[_CHUNK:2 * _CHUNK][None]
    rot_a = consts[2 * _CHUNK:2 * _CHUNK + _NROT][:, None, :]
    rot_b = consts[2 * _CHUNK + _NROT:][:, None, :]
    pe = (pe16 * rot_a + pe16s * rot_b).reshape(_L, _D)
    mesh = plsc.VectorSubcoreMesh(core_axis_name="c", subcore_axis_name="s")
    f = functools.partial(
        pl.kernel,
        mesh=mesh,
        out_type=jax.ShapeDtypeStruct((_ROWS, _D), jnp.float32),
        scratch_types=[
            pltpu.VMEM((_B * _PPW,), jnp.int32),
            pltpu.VMEM((_CHUNK, _D), jnp.float32),
            pltpu.VMEM((_CHUNK, _D), jnp.float32),
            pltpu.VMEM((_CHUNK, _D), jnp.float32),
            pltpu.VMEM((_CHUNK, _D), jnp.float32),
            pltpu.VMEM((_CHUNK, _D), jnp.float32),
            pltpu.VMEM((_CHUNK, _D), jnp.float32),
            pltpu.SemaphoreType.DMA,
            pltpu.SemaphoreType.DMA,
            pltpu.SemaphoreType.DMA,
            pltpu.SemaphoreType.DMA,
            pltpu.SemaphoreType.DMA,
            pltpu.SemaphoreType.DMA,
            pltpu.SemaphoreType.DMA,
            pltpu.SemaphoreType.DMA,
            pltpu.SemaphoreType.DMA,
            pltpu.SemaphoreType.DMA,
        ],
    )(_embed_body)
    return f(x, pe, table)


def kernel(x, table):
    return _embed(x, table).reshape(_B, _L, _D)


# speedup vs baseline: 1.1172x; 1.0166x over previous
import functools
import math

import numpy as np
import jax
import jax.numpy as jnp
from jax import lax
from jax.experimental import pallas as pl
from jax.experimental.pallas import tpu as pltpu
from jax.experimental.pallas import tpu_sc as plsc

_VOCAB = 100000
_D = 1024
_B = 4
_L = 2048
_NC, _NS = 2, 16
_NW = _NC * _NS
_PPW = _L // _NW
_ROWS = _B * _L
_CHUNK = 16
_NGROUP = _PPW // _CHUNK
_NCHUNK = _NGROUP * _B
_LANES = 16
_NBUF = 4


_NROT = _L // _CHUNK


def _pe_consts() -> np.ndarray:
    div = np.exp(np.arange(0, _D, 2).astype(np.float32)
                 * (-math.log(10000.0) / _D))
    pos = np.arange(_CHUNK, dtype=np.float32)[:, None]
    pe16 = np.zeros((_CHUNK, _D), dtype=np.float32)
    pe16[:, 0::2] = np.sin(pos * div)
    pe16[:, 1::2] = np.cos(pos * div)
    pe16s = np.zeros_like(pe16)
    pe16s[:, 0::2] = pe16[:, 1::2]
    pe16s[:, 1::2] = pe16[:, 0::2]
    k = (np.arange(_NROT, dtype=np.float32) * _CHUNK)[:, None]
    rot_a = np.zeros((_NROT, _D), dtype=np.float32)
    rot_a[:, 0::2] = np.cos(k * div)
    rot_a[:, 1::2] = rot_a[:, 0::2]
    rot_b = np.zeros((_NROT, _D), dtype=np.float32)
    rot_b[:, 0::2] = np.sin(k * div)
    rot_b[:, 1::2] = -rot_b[:, 0::2]
    return np.concatenate([pe16, pe16s, rot_a, rot_b], axis=0)


_CONSTS = _pe_consts()


def _embed_body(x_hbm, pe_hbm, table_hbm, out_hbm,
                idx_v, pv0, pv1, gb0, gb1, gb2, gb3,
                psem0, psem1, gsem0, gsem1, gsem2, gsem3,
                ssem0, ssem1, ssem2, ssem3):
    pe_v = (pv0, pv1)
    gbuf = (gb0, gb1, gb2, gb3)
    psem = (psem0, psem1)
    gsem = (gsem0, gsem1, gsem2, gsem3)
    ssem = (ssem0, ssem1, ssem2, ssem3)

    wid = lax.axis_index("s") * _NC + lax.axis_index("c")
    p0 = wid * _PPW

    for b in range(_B):
        pltpu.sync_copy(x_hbm.at[b, pl.ds(p0, _PPW)],
                        idx_v.at[pl.ds(b * _PPW, _PPW)])

    def start_gather(c):
        g, bt = divmod(c, _B)
        return pltpu.async_copy(
            table_hbm.at[idx_v.at[pl.ds(bt * _PPW + g * _CHUNK, _CHUNK)]],
            gbuf[c % _NBUF], gsem[c % _NBUF])

    def start_pe(g):
        return pltpu.async_copy(
            pe_hbm.at[pl.ds(p0 + g * _CHUNK, _CHUNK)], pe_v[g % 2],
            psem[g % 2])

    gathers = {c: start_gather(c) for c in range(_NBUF - 1)}
    pes = {0: start_pe(0), 1: start_pe(1)}
    stores = {}

    for c in range(_NCHUNK):
        g, bt = divmod(c, _B)
        if bt == 0:
            if 1 <= g <= _NGROUP - 2:
                pes[g + 1] = start_pe(g + 1)
            pes.pop(g).wait()
        gathers.pop(c).wait()

        gb, pv = gbuf[c % _NBUF], pe_v[g % 2]

        @plsc.parallel_loop(0, _CHUNK * _D // (2 * _LANES), 1, unroll=4)
        def add_body(i):
            r = lax.shift_right_logical(i, 5)
            c0 = pl.multiple_of(
                lax.bitwise_and(i, _D // (2 * _LANES) - 1) * 2 * _LANES,
                2 * _LANES)
            s1 = pl.ds(c0, _LANES)
            s2 = pl.ds(c0 + _LANES, _LANES)
            plsc.addupdate(gb.at[r, s1], pv[r, s1])
            plsc.addupdate(gb.at[r, s2], pv[r, s2])

        stores[c] = pltpu.async_copy(
            gb, out_hbm.at[pl.ds(bt * _L + p0 + g * _CHUNK, _CHUNK)],
            ssem[c % _NBUF])
        if c + _NBUF - 1 < _NCHUNK:
            if c >= 1:
                stores.pop(c - 1).wait()
            gathers[c + _NBUF - 1] = start_gather(c + _NBUF - 1)

    for c in sorted(stores):
        stores.pop(c).wait()


@jax.jit
def _embed(x, table):
    anchor = jnp.float32(0) * x[0, 0].astype(jnp.float32)
    consts = jnp.asarray(_CONSTS)
    pe16 = consts[0:_CHUNK][None]
    pe16s = consts[_CHUNK:2 * _CHUNK][None]
    rot_a = consts[2 * _CHUNK:2 * _CHUNK + _NROT][:, None, :]
    rot_b = consts[2 * _CHUNK + _NROT:][:, None, :]
    pe = (pe16 * rot_a + pe16s * rot_b + anchor).reshape(_L, _D)
    mesh = plsc.VectorSubcoreMesh(core_axis_name="c", subcore_axis_name="s")
    f = functools.partial(
        pl.kernel,
        mesh=mesh,
        out_type=jax.ShapeDtypeStruct((_ROWS, _D), jnp.float32),
        scratch_types=[
            pltpu.VMEM((_B * _PPW,), jnp.int32),
            pltpu.VMEM((_CHUNK, _D), jnp.float32),
            pltpu.VMEM((_CHUNK, _D), jnp.float32),
            pltpu.VMEM((_CHUNK, _D), jnp.float32),
            pltpu.VMEM((_CHUNK, _D), jnp.float32),
            pltpu.VMEM((_CHUNK, _D), jnp.float32),
            pltpu.VMEM((_CHUNK, _D), jnp.float32),
            pltpu.SemaphoreType.DMA,
            pltpu.SemaphoreType.DMA,
            pltpu.SemaphoreType.DMA,
            pltpu.SemaphoreType.DMA,
            pltpu.SemaphoreType.DMA,
            pltpu.SemaphoreType.DMA,
            pltpu.SemaphoreType.DMA,
            pltpu.SemaphoreType.DMA,
            pltpu.SemaphoreType.DMA,
            pltpu.SemaphoreType.DMA,
        ],
    )(_embed_body)
    return f(x, pe, table)


def kernel(x, table):
    return _embed(x, table).reshape(_B, _L, _D)


# speedup vs baseline: 1.1190x; 1.0016x over previous
import functools
import math

import numpy as np
import jax
import jax.numpy as jnp
from jax import lax
from jax.experimental import pallas as pl
from jax.experimental.pallas import tpu as pltpu
from jax.experimental.pallas import tpu_sc as plsc

_VOCAB = 100000
_D = 1024
_B = 4
_L = 2048
_NC, _NS = 2, 16
_NW = _NC * _NS
_PPW = _L // _NW
_ROWS = _B * _L
_CHUNK = 16
_NGROUP = _PPW // _CHUNK
_NCHUNK = _NGROUP * _B
_LANES = 16
_NBUF = 5


_NROT = _L // _CHUNK


def _pe_consts() -> np.ndarray:
    div = np.exp(np.arange(0, _D, 2).astype(np.float32)
                 * (-math.log(10000.0) / _D))
    pos = np.arange(_CHUNK, dtype=np.float32)[:, None]
    pe16 = np.zeros((_CHUNK, _D), dtype=np.float32)
    pe16[:, 0::2] = np.sin(pos * div)
    pe16[:, 1::2] = np.cos(pos * div)
    pe16s = np.zeros_like(pe16)
    pe16s[:, 0::2] = pe16[:, 1::2]
    pe16s[:, 1::2] = pe16[:, 0::2]
    k = (np.arange(_NROT, dtype=np.float32) * _CHUNK)[:, None]
    rot_a = np.zeros((_NROT, _D), dtype=np.float32)
    rot_a[:, 0::2] = np.cos(k * div)
    rot_a[:, 1::2] = rot_a[:, 0::2]
    rot_b = np.zeros((_NROT, _D), dtype=np.float32)
    rot_b[:, 0::2] = np.sin(k * div)
    rot_b[:, 1::2] = -rot_b[:, 0::2]
    return np.concatenate([pe16, pe16s, rot_a, rot_b], axis=0)


_CONSTS = _pe_consts()


def _embed_body(x_hbm, pe_hbm, table_hbm, out_hbm,
                idx_v, pv0, pv1, gb0, gb1, gb2, gb3, gb4,
                psem0, psem1, gsem0, gsem1, gsem2, gsem3, gsem4,
                ssem0, ssem1, ssem2, ssem3, ssem4):
    pe_v = (pv0, pv1)
    gbuf = (gb0, gb1, gb2, gb3, gb4)
    psem = (psem0, psem1)
    gsem = (gsem0, gsem1, gsem2, gsem3, gsem4)
    ssem = (ssem0, ssem1, ssem2, ssem3, ssem4)

    wid = lax.axis_index("s") * _NC + lax.axis_index("c")
    p0 = wid * _PPW

    for b in range(_B):
        pltpu.sync_copy(x_hbm.at[b, pl.ds(p0, _PPW)],
                        idx_v.at[pl.ds(b * _PPW, _PPW)])

    def start_gather(c):
        g, bt = divmod(c, _B)
        return pltpu.async_copy(
            table_hbm.at[idx_v.at[pl.ds(bt * _PPW + g * _CHUNK, _CHUNK)]],
            gbuf[c % _NBUF], gsem[c % _NBUF])

    def start_pe(g):
        return pltpu.async_copy(
            pe_hbm.at[pl.ds(p0 + g * _CHUNK, _CHUNK)], pe_v[g % 2],
            psem[g % 2])

    gathers = {c: start_gather(c) for c in range(_NBUF - 2)}
    pes = {0: start_pe(0), 1: start_pe(1)}
    stores = {}

    for c in range(_NCHUNK):
        g, bt = divmod(c, _B)
        if bt == 0:
            if 1 <= g <= _NGROUP - 2:
                pes[g + 1] = start_pe(g + 1)
            pes.pop(g).wait()
        gathers.pop(c).wait()

        gb, pv = gbuf[c % _NBUF], pe_v[g % 2]

        @plsc.parallel_loop(0, _CHUNK * _D // (2 * _LANES), 1, unroll=4)
        def add_body(i):
            r = lax.shift_right_logical(i, 5)
            c0 = pl.multiple_of(
                lax.bitwise_and(i, _D // (2 * _LANES) - 1) * 2 * _LANES,
                2 * _LANES)
            s1 = pl.ds(c0, _LANES)
            s2 = pl.ds(c0 + _LANES, _LANES)
            plsc.addupdate(gb.at[r, s1], pv[r, s1])
            plsc.addupdate(gb.at[r, s2], pv[r, s2])

        stores[c] = pltpu.async_copy(
            gb, out_hbm.at[pl.ds(bt * _L + p0 + g * _CHUNK, _CHUNK)],
            ssem[c % _NBUF])
        if c + _NBUF - 2 < _NCHUNK:
            if c >= 2:
                stores.pop(c - 2).wait()
            gathers[c + _NBUF - 2] = start_gather(c + _NBUF - 2)

    for c in sorted(stores):
        stores.pop(c).wait()


@jax.jit
def _embed(x, table):
    anchor = jnp.float32(0) * x[0, 0].astype(jnp.float32)
    consts = jnp.asarray(_CONSTS)
    pe16 = consts[0:_CHUNK][None]
    pe16s = consts[_CHUNK:2 * _CHUNK][None]
    rot_a = consts[2 * _CHUNK:2 * _CHUNK + _NROT][:, None, :]
    rot_b = consts[2 * _CHUNK + _NROT:][:, None, :]
    pe = (pe16 * rot_a + pe16s * rot_b + anchor).reshape(_L, _D)
    mesh = plsc.VectorSubcoreMesh(core_axis_name="c", subcore_axis_name="s")
    f = functools.partial(
        pl.kernel,
        mesh=mesh,
        out_type=jax.ShapeDtypeStruct((_ROWS, _D), jnp.float32),
        scratch_types=[
            pltpu.VMEM((_B * _PPW,), jnp.int32),
            pltpu.VMEM((_CHUNK, _D), jnp.float32),
            pltpu.VMEM((_CHUNK, _D), jnp.float32),
            pltpu.VMEM((_CHUNK, _D), jnp.float32),
            pltpu.VMEM((_CHUNK, _D), jnp.float32),
            pltpu.VMEM((_CHUNK, _D), jnp.float32),
            pltpu.VMEM((_CHUNK, _D), jnp.float32),
            pltpu.VMEM((_CHUNK, _D), jnp.float32),
            pltpu.SemaphoreType.DMA,
            pltpu.SemaphoreType.DMA,
            pltpu.SemaphoreType.DMA,
            pltpu.SemaphoreType.DMA,
            pltpu.SemaphoreType.DMA,
            pltpu.SemaphoreType.DMA,
            pltpu.SemaphoreType.DMA,
            pltpu.SemaphoreType.DMA,
            pltpu.SemaphoreType.DMA,
            pltpu.SemaphoreType.DMA,
            pltpu.SemaphoreType.DMA,
            pltpu.SemaphoreType.DMA,
        ],
    )(_embed_body)
    return f(x, pe, table)


def kernel(x, table):
    return _embed(x, table).reshape(_B, _L, _D)
